# baseline (speedup 1.0000x reference)
import jax
import jax.numpy as jnp
from jax.experimental import pallas as pl

_B, _T, _C = 16, 32, 128
_H, _MID, _DH = 128, 64, 128
_BB = 8


def _seq2seq_kernel(u_ref, c_ref, cpt_ref,
                    W_u_ref, b_u_ref, W_c_ref, b_c_ref, W_e_ref, b_e_ref,
                    W_att_ref, b_att_ref, Wf_u_ref, Wf_c_ref, Wf_o_ref, Wf_ref,
                    out_ref):
    T, C, H, MID, BB = _T, _C, _H, _MID, _BB
    R = BB * T
    u = u_ref[...].reshape(R, 2 * H)
    c = c_ref[...].reshape(R, _DH)

    res_u = jnp.dot(u, W_u_ref[...], preferred_element_type=jnp.float32) + b_u_ref[...]
    res_c = jnp.dot(c, W_c_ref[...], preferred_element_type=jnp.float32) + b_c_ref[...]
    s_uc = jnp.dot(res_u + res_c, W_att_ref[...],
                   preferred_element_type=jnp.float32) + b_att_ref[...]

    v_att = jnp.dot(W_e_ref[...], W_att_ref[...],
                    preferred_element_type=jnp.float32)
    wfo_f = jnp.dot(Wf_o_ref[...], Wf_ref[...],
                    preferred_element_type=jnp.float32)
    be_att = jnp.dot(b_e_ref[...], W_att_ref[...],
                     preferred_element_type=jnp.float32)
    v2t = jnp.concatenate([v_att, wfo_f], axis=1).T

    cpt2 = cpt_ref[...].reshape(R * C, H)
    P = jax.lax.dot_general(v2t, cpt2, (((1,), (1,)), ((), ())),
                            preferred_element_type=jnp.float32)
    s_e = P[0:1, :].reshape(R, C)
    q = P[1:2, :].reshape(R, C)

    scores = s_e + (s_uc + be_att)
    mx = jnp.max(scores, axis=1, keepdims=True)
    ex = jnp.exp(scores - mx)
    dist = ex / jnp.sum(ex, axis=1, keepdims=True)

    res_f_uc = (jnp.dot(u, Wf_u_ref[...], preferred_element_type=jnp.float32)
                + jnp.dot(c, Wf_c_ref[...], preferred_element_type=jnp.float32))
    glogit = (jnp.dot(res_f_uc, Wf_ref[...], preferred_element_type=jnp.float32)
              + jnp.sum(dist * q, axis=1, keepdims=True))
    g = jax.nn.sigmoid(glogit)

    t_idx = jax.lax.broadcasted_iota(jnp.int32, (R, 1), 0)
    g_eff = jnp.where(t_idx % T == 0, 0.0, g)
    lg = jnp.log(g)
    row = jax.lax.broadcasted_iota(jnp.int32, (R, R), 0)
    col = jax.lax.broadcasted_iota(jnp.int32, (R, R), 1)
    umask = ((col > row) & (col // T == row // T)).astype(jnp.float32)
    m = jnp.exp(jnp.dot(umask, lg, preferred_element_type=jnp.float32))
    w = (1.0 - g_eff) * m

    out_ref[...] = (dist * w).reshape(BB, T, C)


def kernel(utterance, dialog, cpt_emb, W_u, b_u, W_c, b_c, W_e, b_e,
           W_att, b_att, Wf_u, Wf_c, Wf_o, Wf):
    B, T, C, H = cpt_emb.shape
    MID = W_u.shape[1]
    DH = dialog.shape[2]
    BB = _BB

    c_shift = jnp.concatenate(
        [jnp.zeros_like(dialog[:, :1]), dialog[:, :-1]], axis=1)

    b_u2 = b_u.reshape(1, MID)
    b_c2 = b_c.reshape(1, MID)
    b_e2 = b_e.reshape(1, MID)
    b_att2 = b_att.reshape(1, 1)

    full = lambda shape: pl.BlockSpec(shape, lambda b: (0,) * len(shape))

    out = pl.pallas_call(
        _seq2seq_kernel,
        grid=(B // BB,),
        in_specs=[
            pl.BlockSpec((BB, T, 2 * H), lambda b: (b, 0, 0)),
            pl.BlockSpec((BB, T, DH), lambda b: (b, 0, 0)),
            pl.BlockSpec((BB, T, C, H), lambda b: (b, 0, 0, 0)),
            full((2 * H, MID)), full((1, MID)),
            full((DH, MID)), full((1, MID)),
            full((H, MID)), full((1, MID)),
            full((MID, 1)), full((1, 1)),
            full((2 * H, MID)), full((DH, MID)), full((H, MID)),
            full((MID, 1)),
        ],
        out_specs=pl.BlockSpec((BB, T, C), lambda b: (b, 0, 0)),
        out_shape=jax.ShapeDtypeStruct((B, T, C), jnp.float32),
    )(utterance, c_shift, cpt_emb,
      W_u, b_u2, W_c, b_c2, W_e, b_e2, W_att, b_att2,
      Wf_u, Wf_c, Wf_o, Wf)

    return out.reshape(B, T * C)


# speedup vs baseline: 1.2552x; 1.2552x over previous
import jax
import jax.numpy as jnp
from jax.experimental import pallas as pl

_B, _T, _C = 16, 32, 128
_H, _MID, _DH = 128, 64, 128
_BB = 4


def _seq2seq_kernel(u_ref, d_ref, cpt_ref,
                    W_e_ref, W_att_ref, Wf_u_ref, Wf_c_ref, Wf_o_ref, Wf_ref,
                    out_ref):
    T, C, H, BB, DH = _T, _C, _H, _BB, _DH
    R = BB * T
    u = u_ref[...].reshape(R, 2 * H)
    d = d_ref[...]
    c = jnp.concatenate(
        [jnp.zeros((BB, 1, DH), jnp.float32), d[:, :T - 1, :]],
        axis=1).reshape(R, DH)

    v_att = jnp.dot(W_e_ref[...], W_att_ref[...],
                    preferred_element_type=jnp.float32)
    wfo_f = jnp.dot(Wf_o_ref[...], Wf_ref[...],
                    preferred_element_type=jnp.float32)
    wfu_f = jnp.dot(Wf_u_ref[...], Wf_ref[...],
                    preferred_element_type=jnp.float32)
    wfc_f = jnp.dot(Wf_c_ref[...], Wf_ref[...],
                    preferred_element_type=jnp.float32)
    v2t = jnp.concatenate([v_att, wfo_f], axis=1).T

    cpt2 = cpt_ref[...].reshape(R * C, H)
    P = jax.lax.dot_general(v2t, cpt2, (((1,), (1,)), ((), ())),
                            preferred_element_type=jnp.float32)
    scores = P[0:1, :].reshape(R, C)
    q = P[1:2, :].reshape(R, C)

    mx = jnp.max(scores, axis=1, keepdims=True)
    ex = jnp.exp(scores - mx)
    dist = ex / jnp.sum(ex, axis=1, keepdims=True)

    glogit = (jnp.dot(u, wfu_f, preferred_element_type=jnp.float32)
              + jnp.dot(c, wfc_f, preferred_element_type=jnp.float32)
              + jnp.sum(dist * q, axis=1, keepdims=True))
    g = jax.nn.sigmoid(glogit)

    t_idx = jax.lax.broadcasted_iota(jnp.int32, (R, 1), 0)
    g_eff = jnp.where(t_idx % T == 0, 0.0, g)
    lg = jnp.log(g)
    row = jax.lax.broadcasted_iota(jnp.int32, (R, R), 0)
    col = jax.lax.broadcasted_iota(jnp.int32, (R, R), 1)
    umask = ((col > row) & (col // T == row // T)).astype(jnp.float32)
    m = jnp.exp(jnp.dot(umask, lg, preferred_element_type=jnp.float32))
    w = (1.0 - g_eff) * m

    out_ref[...] = (dist * w).reshape(BB, T, C)


def kernel(utterance, dialog, cpt_emb, W_u, b_u, W_c, b_c, W_e, b_e,
           W_att, b_att, Wf_u, Wf_c, Wf_o, Wf):
    B, T, C, H = cpt_emb.shape
    MID = W_e.shape[1]
    DH = dialog.shape[2]
    BB = _BB

    full = lambda shape: pl.BlockSpec(shape, lambda b: (0,) * len(shape))

    out = pl.pallas_call(
        _seq2seq_kernel,
        grid=(B // BB,),
        in_specs=[
            pl.BlockSpec((BB, T, 2 * H), lambda b: (b, 0, 0)),
            pl.BlockSpec((BB, T, DH), lambda b: (b, 0, 0)),
            pl.BlockSpec((BB, T, C, H), lambda b: (b, 0, 0, 0)),
            full((H, MID)), full((MID, 1)),
            full((2 * H, MID)), full((DH, MID)), full((H, MID)),
            full((MID, 1)),
        ],
        out_specs=pl.BlockSpec((BB, T, C), lambda b: (b, 0, 0)),
        out_shape=jax.ShapeDtypeStruct((B, T, C), jnp.float32),
    )(utterance, dialog, cpt_emb,
      W_e, W_att, Wf_u, Wf_c, Wf_o, Wf)

    return out.reshape(B, T * C)


# speedup vs baseline: 1.6769x; 1.3359x over previous
import jax
import jax.numpy as jnp
from jax.experimental import pallas as pl

_B, _T, _C = 16, 32, 128
_H, _MID, _DH = 128, 64, 128
_BB = 4


def _seq2seq_kernel(u_ref, d_ref, cpt_ref, wp_ref, out_ref):
    T, C, H, BB, DH = _T, _C, _H, _BB, _DH
    R = BB * T
    u = u_ref[...].reshape(R, 2 * H)
    d = d_ref[...]
    c = jnp.concatenate(
        [jnp.zeros((BB, 1, DH), jnp.float32), d[:, :T - 1, :]],
        axis=1).reshape(R, DH)

    v_att = wp_ref[0:H, :]
    wfo_f = wp_ref[H:2 * H, :]
    wfu_f = wp_ref[2 * H:4 * H, :]
    wfc_f = wp_ref[4 * H:5 * H, :]
    v2t = jnp.concatenate([v_att, wfo_f], axis=1).T

    cpt2 = cpt_ref[...].reshape(R * C, H)
    P = jax.lax.dot_general(v2t, cpt2, (((1,), (1,)), ((), ())),
                            preferred_element_type=jnp.float32)
    scores = P[0:1, :].reshape(R, C)
    q = P[1:2, :].reshape(R, C)

    mx = jnp.max(scores, axis=1, keepdims=True)
    ex = jnp.exp(scores - mx)
    dist = ex / jnp.sum(ex, axis=1, keepdims=True)

    glogit = (jnp.dot(u, wfu_f, preferred_element_type=jnp.float32)
              + jnp.dot(c, wfc_f, preferred_element_type=jnp.float32)
              + jnp.sum(dist * q, axis=1, keepdims=True))
    g = jax.nn.sigmoid(glogit)

    t_idx = jax.lax.broadcasted_iota(jnp.int32, (R, 1), 0)
    g_eff = jnp.where(t_idx % T == 0, 0.0, g)
    lg = jnp.log(g)
    row = jax.lax.broadcasted_iota(jnp.int32, (R, R), 0)
    col = jax.lax.broadcasted_iota(jnp.int32, (R, R), 1)
    umask = ((col > row) & (col // T == row // T)).astype(jnp.float32)
    m = jnp.exp(jnp.dot(umask, lg, preferred_element_type=jnp.float32))
    w = (1.0 - g_eff) * m

    out_ref[...] = (dist * w).reshape(BB, T, C)


def kernel(utterance, dialog, cpt_emb, W_u, b_u, W_c, b_c, W_e, b_e,
           W_att, b_att, Wf_u, Wf_c, Wf_o, Wf):
    B, T, C, H = cpt_emb.shape
    MID = W_e.shape[1]
    DH = dialog.shape[2]
    BB = _BB

    wf_all = jnp.dot(jnp.concatenate([Wf_o, Wf_u, Wf_c], axis=0), Wf)
    wpack = jnp.concatenate([jnp.dot(W_e, W_att), wf_all], axis=0)

    out = pl.pallas_call(
        _seq2seq_kernel,
        grid=(B // BB,),
        in_specs=[
            pl.BlockSpec((BB, T, 2 * H), lambda b: (b, 0, 0)),
            pl.BlockSpec((BB, T, DH), lambda b: (b, 0, 0)),
            pl.BlockSpec((BB, T, C, H), lambda b: (b, 0, 0, 0)),
            pl.BlockSpec((5 * H, 1), lambda b: (0, 0)),
        ],
        out_specs=pl.BlockSpec((BB, T, C), lambda b: (b, 0, 0)),
        out_shape=jax.ShapeDtypeStruct((B, T, C), jnp.float32),
    )(utterance, dialog, cpt_emb, wpack)

    return out.reshape(B, T * C)


# speedup vs baseline: 1.7930x; 1.0692x over previous
import jax
import jax.numpy as jnp
from jax.experimental import pallas as pl

_B, _T, _C = 16, 32, 128
_H, _MID, _DH = 128, 64, 128
_BB = 4


def _seq2seq_kernel(u_ref, d_ref, cpt_ref, wp_ref, out_ref):
    T, C, H, BB, DH = _T, _C, _H, _BB, _DH
    R = BB * T
    u = u_ref[...].reshape(R, 2 * H)
    d = d_ref[...]
    c = jnp.concatenate(
        [jnp.zeros((BB, 1, DH), jnp.float32), d[:, :T - 1, :]],
        axis=1).reshape(R, DH)

    v_att = wp_ref[0:H, 0:1]
    wfo_f = wp_ref[H:2 * H, 1:2]
    wfu_f = wp_ref[2 * H:4 * H, 1:2]
    wfc_f = wp_ref[4 * H:5 * H, 1:2]
    v2t = jnp.concatenate([v_att, wfo_f], axis=1).T

    cpt2 = cpt_ref[...].reshape(R * C, H)
    P = jax.lax.dot_general(v2t, cpt2, (((1,), (1,)), ((), ())),
                            preferred_element_type=jnp.float32)
    scores = P[0:1, :].reshape(R, C)
    q = P[1:2, :].reshape(R, C)

    mx = jnp.max(scores, axis=1, keepdims=True)
    ex = jnp.exp(scores - mx)
    dist = ex / jnp.sum(ex, axis=1, keepdims=True)

    glogit = (jnp.dot(u, wfu_f, preferred_element_type=jnp.float32)
              + jnp.dot(c, wfc_f, preferred_element_type=jnp.float32)
              + jnp.sum(dist * q, axis=1, keepdims=True))
    g = jax.nn.sigmoid(glogit)

    t_idx = jax.lax.broadcasted_iota(jnp.int32, (R, 1), 0)
    g_eff = jnp.where(t_idx % T == 0, 0.0, g)
    lg = jnp.log(g)
    row = jax.lax.broadcasted_iota(jnp.int32, (R, R), 0)
    col = jax.lax.broadcasted_iota(jnp.int32, (R, R), 1)
    umask = ((col > row) & (col // T == row // T)).astype(jnp.float32)
    m = jnp.exp(jnp.dot(umask, lg, preferred_element_type=jnp.float32))
    w = (1.0 - g_eff) * m

    out_ref[...] = (dist * w).reshape(BB, T, C)


def kernel(utterance, dialog, cpt_emb, W_u, b_u, W_c, b_c, W_e, b_e,
           W_att, b_att, Wf_u, Wf_c, Wf_o, Wf):
    B, T, C, H = cpt_emb.shape
    MID = W_e.shape[1]
    DH = dialog.shape[2]
    BB = _BB

    wpack = jnp.dot(jnp.concatenate([W_e, Wf_o, Wf_u, Wf_c], axis=0),
                    jnp.concatenate([W_att, Wf], axis=1))

    out = pl.pallas_call(
        _seq2seq_kernel,
        grid=(B // BB,),
        in_specs=[
            pl.BlockSpec((BB, T, 2 * H), lambda b: (b, 0, 0)),
            pl.BlockSpec((BB, T, DH), lambda b: (b, 0, 0)),
            pl.BlockSpec((BB, T, C, H), lambda b: (b, 0, 0, 0)),
            pl.BlockSpec((5 * H, 2), lambda b: (0, 0)),
        ],
        out_specs=pl.BlockSpec((BB, T, C), lambda b: (b, 0, 0)),
        out_shape=jax.ShapeDtypeStruct((B, T, C), jnp.float32),
    )(utterance, dialog, cpt_emb, wpack)

    return out.reshape(B, T * C)


# speedup vs baseline: 1.8251x; 1.0179x over previous
import jax
import jax.numpy as jnp
from jax.experimental import pallas as pl

_B, _T, _C = 16, 32, 128
_H, _MID, _DH = 128, 64, 128
_BB = 4


def _seq2seq_kernel(u_ref, d_ref, cpt_ref, wp_ref, out_ref):
    T, C, H, BB, DH = _T, _C, _H, _BB, _DH
    R = BB * T
    u = u_ref[...].reshape(R, 2 * H)
    d = d_ref[...]
    c = jnp.concatenate(
        [jnp.zeros((BB, 1, DH), jnp.float32), d[:, :T - 1, :]],
        axis=1).reshape(R, DH)

    v2t = jnp.concatenate(
        [wp_ref[0:1, 0:H], wp_ref[1:2, H:2 * H]], axis=0)
    wfu_row = wp_ref[1:2, 2 * H:4 * H]
    wfc_row = wp_ref[1:2, 4 * H:5 * H]

    cpt2 = cpt_ref[...].reshape(R * C, H)
    P = jax.lax.dot_general(v2t, cpt2, (((1,), (1,)), ((), ())),
                            preferred_element_type=jnp.float32)
    scores = P[0:1, :].reshape(R, C)
    q = P[1:2, :].reshape(R, C)

    mx = jnp.max(scores, axis=1, keepdims=True)
    ex = jnp.exp(scores - mx)
    dist = ex / jnp.sum(ex, axis=1, keepdims=True)

    gl_row = (jax.lax.dot_general(wfu_row, u, (((1,), (1,)), ((), ())),
                                  preferred_element_type=jnp.float32)
              + jax.lax.dot_general(wfc_row, c, (((1,), (1,)), ((), ())),
                                    preferred_element_type=jnp.float32))
    glogit = gl_row.T + jnp.sum(dist * q, axis=1, keepdims=True)
    g = jax.nn.sigmoid(glogit)

    t_idx = jax.lax.broadcasted_iota(jnp.int32, (R, 1), 0)
    g_eff = jnp.where(t_idx % T == 0, 0.0, g)
    lg = jnp.log(g)
    row = jax.lax.broadcasted_iota(jnp.int32, (R, R), 0)
    col = jax.lax.broadcasted_iota(jnp.int32, (R, R), 1)
    umask = ((col > row) & (col // T == row // T)).astype(jnp.float32)
    m = jnp.exp(jnp.dot(umask, lg, preferred_element_type=jnp.float32))
    w = (1.0 - g_eff) * m

    out_ref[...] = (dist * w).reshape(BB, T, C)


def kernel(utterance, dialog, cpt_emb, W_u, b_u, W_c, b_c, W_e, b_e,
           W_att, b_att, Wf_u, Wf_c, Wf_o, Wf):
    B, T, C, H = cpt_emb.shape
    MID = W_e.shape[1]
    DH = dialog.shape[2]
    BB = _BB

    wpack = jax.lax.dot_general(
        jnp.concatenate([W_att, Wf], axis=1),
        jnp.concatenate([W_e, Wf_o, Wf_u, Wf_c], axis=0),
        (((0,), (1,)), ((), ())))

    out = pl.pallas_call(
        _seq2seq_kernel,
        grid=(B // BB,),
        in_specs=[
            pl.BlockSpec((BB, T, 2 * H), lambda b: (b, 0, 0)),
            pl.BlockSpec((BB, T, DH), lambda b: (b, 0, 0)),
            pl.BlockSpec((BB, T, C, H), lambda b: (b, 0, 0, 0)),
            pl.BlockSpec((2, 5 * H), lambda b: (0, 0)),
        ],
        out_specs=pl.BlockSpec((BB, T, C), lambda b: (b, 0, 0)),
        out_shape=jax.ShapeDtypeStruct((B, T, C), jnp.float32),
    )(utterance, dialog, cpt_emb, wpack)

    return out.reshape(B, T * C)


# speedup vs baseline: 1.8352x; 1.0055x over previous
import jax
import jax.numpy as jnp
from jax.experimental import pallas as pl

_B, _T, _C = 16, 32, 128
_H, _MID, _DH = 128, 64, 128
_BB = 4


def _seq2seq_kernel(u_ref, d_ref, cpt_ref, wp_ref, out_ref):
    T, C, H, BB, DH = _T, _C, _H, _BB, _DH
    R = BB * T
    u = u_ref[...].reshape(R, 2 * H)
    d = d_ref[...]
    c = jnp.concatenate(
        [jnp.zeros((BB, 1, DH), jnp.float32), d[:, :T - 1, :]],
        axis=1).reshape(R, DH)

    v2t = jnp.concatenate(
        [wp_ref[0:1, 0:H], wp_ref[1:2, H:2 * H]], axis=0)
    wfu_row = wp_ref[1:2, 2 * H:4 * H]
    wfc_row = wp_ref[1:2, 4 * H:5 * H]

    cpt2 = cpt_ref[...].reshape(R * C, H)
    P = jax.lax.dot_general(v2t, cpt2, (((1,), (1,)), ((), ())),
                            preferred_element_type=jnp.float32)
    scores = P[0:1, :].reshape(R, C)
    q = P[1:2, :].reshape(R, C)

    mx = jnp.max(scores, axis=1, keepdims=True)
    ex = jnp.exp(scores - mx)
    dist = ex / jnp.sum(ex, axis=1, keepdims=True)

    gl_row = (jax.lax.dot_general(wfu_row, u, (((1,), (1,)), ((), ())),
                                  preferred_element_type=jnp.float32)
              + jax.lax.dot_general(wfc_row, c, (((1,), (1,)), ((), ())),
                                    preferred_element_type=jnp.float32))
    glogit = gl_row.T + jnp.sum(dist * q, axis=1, keepdims=True)
    g = jax.nn.sigmoid(glogit)

    t_idx = jax.lax.broadcasted_iota(jnp.int32, (R, 1), 0)
    g_eff = jnp.where(t_idx % T == 0, 0.0, g)
    lg = jnp.log(g)
    row = jax.lax.broadcasted_iota(jnp.int32, (R, R), 0)
    col = jax.lax.broadcasted_iota(jnp.int32, (R, R), 1)
    umask = ((col > row) & (col // T == row // T)).astype(jnp.float32)
    m = jnp.exp(jnp.dot(umask, lg, preferred_element_type=jnp.float32))
    w = (1.0 - g_eff) * m

    out_ref[...] = (dist * w).reshape(1, BB, T * C)


def kernel(utterance, dialog, cpt_emb, W_u, b_u, W_c, b_c, W_e, b_e,
           W_att, b_att, Wf_u, Wf_c, Wf_o, Wf):
    B, T, C, H = cpt_emb.shape
    MID = W_e.shape[1]
    DH = dialog.shape[2]
    BB = _BB

    wpack = jax.lax.dot_general(
        jnp.concatenate([W_att, Wf], axis=1),
        jnp.concatenate([W_e, Wf_o, Wf_u, Wf_c], axis=0),
        (((0,), (1,)), ((), ())))

    out = pl.pallas_call(
        _seq2seq_kernel,
        grid=(B // BB,),
        in_specs=[
            pl.BlockSpec((BB, T, 2 * H), lambda b: (b, 0, 0)),
            pl.BlockSpec((BB, T, DH), lambda b: (b, 0, 0)),
            pl.BlockSpec((BB, T, C, H), lambda b: (b, 0, 0, 0)),
            pl.BlockSpec((2, 5 * H), lambda b: (0, 0)),
        ],
        out_specs=pl.BlockSpec((1, BB, T * C), lambda b: (b, 0, 0)),
        out_shape=jax.ShapeDtypeStruct((B // BB, BB, T * C), jnp.float32),
    )(utterance, dialog, cpt_emb, wpack)

    return out.reshape(B, T * C)
